# Initial kernel scaffold; baseline (speedup 1.0000x reference)
import jax
import jax.numpy as jnp
from jax.experimental import pallas as pl


def kernel(coords, angles, theta0, k):
    raise NotImplementedError("write your pallas kernel here")


# speedup vs baseline: 5.8980x; 5.8980x over previous
import functools
import math

import jax
import jax.numpy as jnp
from jax import lax
from jax.experimental import pallas as pl
from jax.experimental.pallas import tpu as pltpu
from jax.experimental.pallas import tpu_sc as plsc

_CUBIC = -0.014
_QUARTIC = 5.6e-05
_PENTIC = -7e-07
_SEXTIC = 2.2e-08

_NC = 2
_NS = 16
_NW = _NC * _NS
_L = 16
_SUB = 128

_PI = math.pi
_A0 = 1.5707288
_A1 = -0.2121144
_A2 = 0.0742610
_A3 = -0.0187293


def _rsqrt(x):
    i = plsc.bitcast(x, jnp.int32)
    i = jnp.int32(0x5F3759DF) - lax.shift_right_logical(i, 1)
    y = plsc.bitcast(i, jnp.float32)
    for _ in range(3):
        y = y * (1.5 - 0.5 * x * y * y)
    return y


def _arccos(x):
    a = jnp.abs(x)
    p = ((_A3 * a + _A2) * a + _A1) * a + _A0
    u = 1.0 - a
    s = u * _rsqrt(jnp.maximum(u, 1e-30))
    r = s * p
    return jnp.where(x >= 0.0, r, _PI - r)


def _make_kernel(nsub):
    mesh = plsc.VectorSubcoreMesh(core_axis_name="c", subcore_axis_name="s")
    idx_t = pltpu.VMEM((nsub, _SUB), jnp.int32)
    par_t = pltpu.VMEM((nsub, _SUB), jnp.float32)
    row_t = pltpu.VMEM((_SUB,), jnp.float32)

    @functools.partial(
        pl.kernel,
        out_type=jax.ShapeDtypeStruct((_NW, _L), jnp.float32),
        mesh=mesh,
        compiler_params=pltpu.CompilerParams(needs_layout_passes=False),
        scratch_types=[
            [idx_t] * 9,
            par_t,
            par_t,
            [row_t] * 9,
            pltpu.VMEM((_L,), jnp.float32),
            pltpu.SemaphoreType.DMA,
        ],
    )
    def angle_energy(coords_hbm, idx_hbm, t0_hbm, kk_hbm,
                     out_hbm, idx_v, t0_v, kk_v, comp_v, acc_v, sem):
        w = lax.axis_index("s") * _NC + lax.axis_index("c")
        for n in range(9):
            pltpu.sync_copy(idx_hbm[n].at[w], idx_v[n])
        pltpu.sync_copy(t0_hbm.at[w], t0_v)
        pltpu.sync_copy(kk_hbm.at[w], kk_v)

        def sub(si, acc):
            cps = [
                pltpu.async_copy(coords_hbm.at[idx_v[n].at[si]], comp_v[n], sem)
                for n in range(9)
            ]
            for cp in cps:
                cp.wait()
            for g in range(_SUB // _L):
                sl = pl.ds(g * _L, _L)
                xi, yi, zi = comp_v[0][sl], comp_v[1][sl], comp_v[2][sl]
                xj, yj, zj = comp_v[3][sl], comp_v[4][sl], comp_v[5][sl]
                xk, yk, zk = comp_v[6][sl], comp_v[7][sl], comp_v[8][sl]
                v1x = xi - xj
                v1y = yi - yj
                v1z = zi - zj
                v2x = xk - xj
                v2y = yk - yj
                v2z = zk - zj
                dot = v1x * v2x + v1y * v2y + v1z * v2z
                m1 = v1x * v1x + v1y * v1y + v1z * v1z
                m2 = v2x * v2x + v2y * v2y + v2z * v2z
                cos = dot * _rsqrt(jnp.maximum(m1 * m2, 1e-30))
                cos = jnp.minimum(jnp.maximum(cos, -1.0), 1.0)
                theta = _arccos(cos)
                t0 = t0_v[si, sl]
                kk = kk_v[si, sl]
                dt = theta - t0
                poly = 1.0 + dt * (_CUBIC + dt * (_QUARTIC + dt * (_PENTIC + dt * _SEXTIC)))
                acc = acc + kk * (dt * dt) * poly
            return acc

        acc = lax.fori_loop(0, nsub, sub, jnp.zeros((_L,), jnp.float32))
        acc_v[...] = acc
        pltpu.sync_copy(acc_v, out_hbm.at[w])

    return angle_energy


def kernel(coords, angles, theta0, k):
    m = angles.shape[0]
    group = _NW * _SUB
    mp = ((m + group - 1) // group) * group
    nsub = mp // group
    pad = mp - m
    idx = angles.astype(jnp.int32)
    idx = jnp.pad(idx, ((0, pad), (0, 0)))
    t0 = jnp.pad(theta0.astype(jnp.float32), (0, pad))
    kk = jnp.pad(k.astype(jnp.float32), (0, pad))
    coords_flat = coords.astype(jnp.float32).reshape(-1)
    shape3 = (_NW, nsub, _SUB)
    base = idx * 3
    flat_idx = [
        (base[:, p] + d).reshape(shape3) for p in range(3) for d in range(3)
    ]
    t0 = t0.reshape(shape3)
    kk = kk.reshape(shape3)
    partials = _make_kernel(nsub)(coords_flat, flat_idx, t0, kk)
    return jnp.sum(partials)


# speedup vs baseline: 10.1047x; 1.7132x over previous
import functools
import math

import jax
import jax.numpy as jnp
from jax import lax
from jax.experimental import pallas as pl
from jax.experimental.pallas import tpu as pltpu
from jax.experimental.pallas import tpu_sc as plsc

_CUBIC = -0.014
_QUARTIC = 5.6e-05
_PENTIC = -7e-07
_SEXTIC = 2.2e-08

_NC = 2
_NS = 16
_NW = _NC * _NS
_L = 16
_SUB = 128

_PI = math.pi
_A0 = 1.5707288
_A1 = -0.2121144
_A2 = 0.0742610
_A3 = -0.0187293


def _rsqrt(x):
    i = plsc.bitcast(x, jnp.int32)
    i = jnp.int32(0x5F3759DF) - lax.shift_right_logical(i, 1)
    y = plsc.bitcast(i, jnp.float32)
    for _ in range(3):
        y = y * (1.5 - 0.5 * x * y * y)
    return y


def _arccos(x):
    a = jnp.abs(x)
    p = ((_A3 * a + _A2) * a + _A1) * a + _A0
    u = 1.0 - a
    s = u * _rsqrt(jnp.maximum(u, 1e-30))
    r = s * p
    return jnp.where(x >= 0.0, r, _PI - r)


def _make_kernel(nsub, n3):
    mesh = plsc.VectorSubcoreMesh(core_axis_name="c", subcore_axis_name="s")
    idx_t = pltpu.VMEM((nsub, _SUB), jnp.int32)
    par_t = pltpu.VMEM((nsub, _SUB), jnp.float32)
    row_t = pltpu.VMEM((_SUB,), jnp.float32)

    @functools.partial(
        pl.kernel,
        out_type=jax.ShapeDtypeStruct((_NW, _L), jnp.float32),
        mesh=mesh,
        compiler_params=pltpu.CompilerParams(needs_layout_passes=False),
        scratch_types=[
            pltpu.VMEM_SHARED((n3,), jnp.float32),
            [idx_t] * 9,
            par_t,
            par_t,
            [row_t] * 9,
            pltpu.VMEM((_L,), jnp.float32),
            pltpu.SemaphoreType.DMA,
        ],
    )
    def angle_energy(coords_hbm, idx_hbm, t0_hbm, kk_hbm,
                     out_hbm, csh_v, idx_v, t0_v, kk_v, comp_v, acc_v, sem):
        s_id = lax.axis_index("s")
        w = s_id * _NC + lax.axis_index("c")

        @pl.when(s_id == 0)
        def _():
            pltpu.sync_copy(coords_hbm, csh_v)

        for n in range(9):
            pltpu.sync_copy(idx_hbm[n].at[w], idx_v[n])
        pltpu.sync_copy(t0_hbm.at[w], t0_v)
        pltpu.sync_copy(kk_hbm.at[w], kk_v)
        plsc.subcore_barrier()

        def sub(si, acc):
            cps = [
                pltpu.async_copy(csh_v.at[idx_v[n].at[si]], comp_v[n], sem)
                for n in range(9)
            ]
            for cp in cps:
                cp.wait()
            for g in range(_SUB // _L):
                sl = pl.ds(g * _L, _L)
                xi, yi, zi = comp_v[0][sl], comp_v[1][sl], comp_v[2][sl]
                xj, yj, zj = comp_v[3][sl], comp_v[4][sl], comp_v[5][sl]
                xk, yk, zk = comp_v[6][sl], comp_v[7][sl], comp_v[8][sl]
                v1x = xi - xj
                v1y = yi - yj
                v1z = zi - zj
                v2x = xk - xj
                v2y = yk - yj
                v2z = zk - zj
                dot = v1x * v2x + v1y * v2y + v1z * v2z
                m1 = v1x * v1x + v1y * v1y + v1z * v1z
                m2 = v2x * v2x + v2y * v2y + v2z * v2z
                cos = dot * _rsqrt(jnp.maximum(m1 * m2, 1e-30))
                cos = jnp.minimum(jnp.maximum(cos, -1.0), 1.0)
                theta = _arccos(cos)
                t0 = t0_v[si, sl]
                kk = kk_v[si, sl]
                dt = theta - t0
                poly = 1.0 + dt * (_CUBIC + dt * (_QUARTIC + dt * (_PENTIC + dt * _SEXTIC)))
                acc = acc + kk * (dt * dt) * poly
            return acc

        acc = lax.fori_loop(0, nsub, sub, jnp.zeros((_L,), jnp.float32))
        acc_v[...] = acc
        pltpu.sync_copy(acc_v, out_hbm.at[w])

    return angle_energy


def kernel(coords, angles, theta0, k):
    m = angles.shape[0]
    group = _NW * _SUB
    mp = ((m + group - 1) // group) * group
    nsub = mp // group
    pad = mp - m
    idx = angles.astype(jnp.int32)
    idx = jnp.pad(idx, ((0, pad), (0, 0)))
    t0 = jnp.pad(theta0.astype(jnp.float32), (0, pad))
    kk = jnp.pad(k.astype(jnp.float32), (0, pad))
    coords_flat = coords.astype(jnp.float32).reshape(-1)
    shape3 = (_NW, nsub, _SUB)
    base = idx * 3
    flat_idx = [
        (base[:, p] + d).reshape(shape3) for p in range(3) for d in range(3)
    ]
    t0 = t0.reshape(shape3)
    kk = kk.reshape(shape3)
    partials = _make_kernel(nsub, coords_flat.shape[0])(
        coords_flat, flat_idx, t0, kk)
    return jnp.sum(partials)
